# baseline (speedup 1.0000x reference)
import functools

import jax
import jax.numpy as jnp
from jax import lax
from jax.experimental import pallas as pl
from jax.experimental.pallas import tpu as pltpu
from jax.experimental.pallas import tpu_sc as plsc

D = 64
GD = D // 4
H2 = D // 2
NC, NS = 2, 16
NW = NC * NS
CHUNK = 128
W = 128


@functools.lru_cache(maxsize=None)
def _make_gather(B):
    b_per_w = B // NW
    n_chunks = b_per_w // CHUNK
    mesh = plsc.VectorSubcoreMesh(core_axis_name="c", subcore_axis_name="s")

    @functools.partial(
        pl.kernel,
        mesh=mesh,
        out_type=[jax.ShapeDtypeStruct((B, W), jnp.float32) for _ in range(4)],
        scratch_types=[
            pltpu.VMEM((n_chunks, CHUNK), jnp.int32),
            pltpu.VMEM((n_chunks, CHUNK), jnp.int32),
            pltpu.VMEM((n_chunks, CHUNK), jnp.int32),
            pltpu.VMEM((n_chunks, CHUNK), jnp.int32),
            pltpu.VMEM((2, CHUNK, W), jnp.float32),
            pltpu.VMEM((2, CHUNK, W), jnp.float32),
            pltpu.SemaphoreType.DMA,
        ],
    )
    def gather(pu_hbm, pi_hbm, pgu_hbm, pgi_hbm, umlp, imlp, ugmf, igmf,
               uh_out, ih_out, gu_out, gi_out,
               pu_v, pi_v, pgu_v, pgi_v, ba, bb, sem):
        wid = lax.axis_index("s") * NC + lax.axis_index("c")
        base = wid * b_per_w
        pltpu.sync_copy(pu_hbm.at[wid], pu_v)
        pltpu.sync_copy(pi_hbm.at[wid], pi_v)
        pltpu.sync_copy(pgu_hbm.at[wid], pgu_v)
        pltpu.sync_copy(pgi_hbm.at[wid], pgi_v)

        def run_pair(tab_a, idx_a, out_a, tab_b, idx_b, out_b):
            def fire(c, bank):
                pltpu.async_copy(tab_a.at[idx_a.at[c]], ba.at[bank], sem)
                pltpu.async_copy(tab_b.at[idx_b.at[c]], bb.at[bank], sem)

            def drain(bank):
                pltpu.make_async_copy(tab_a.at[idx_a.at[0]], ba.at[bank],
                                      sem).wait()
                pltpu.make_async_copy(tab_a.at[idx_a.at[0]], bb.at[bank],
                                      sem).wait()

            fire(0, 0)
            for c in range(n_chunks):
                bank = c % 2
                drain(bank)
                if c + 1 < n_chunks:
                    fire(c + 1, 1 - bank)
                sl = pl.ds(base + c * CHUNK, CHUNK)
                pltpu.sync_copy(ba.at[bank], out_a.at[sl])
                pltpu.sync_copy(bb.at[bank], out_b.at[sl])

        run_pair(umlp, pu_v, uh_out, imlp, pi_v, ih_out)
        run_pair(ugmf, pgu_v, gu_out, igmf, pgi_v, gi_out)

    return gather


def _mlp_body(uh2, ih2, gu2, gi2, pu, pi, ohu, ohi,
              w1u, w1i, b1, w2, b2, w3, b3, wfg, wfm, out):
    f32 = jnp.float32
    uh = jnp.where(pu[...] > 0, uh2[:, D:], uh2[:, :D])
    ih = jnp.where(pi[...] > 0, ih2[:, D:], ih2[:, :D])
    h1 = jnp.dot(uh, w1u[...], preferred_element_type=f32)
    h1 += jnp.dot(ih, w1i[...], preferred_element_type=f32) + b1[...]
    h1 = jnp.maximum(h1, 0.0)
    h2 = jnp.maximum(jnp.dot(h1, w2[...], preferred_element_type=f32) + b2[...], 0.0)
    mlp = jnp.dot(h2, w3[...], preferred_element_type=f32) + b3[...]
    gu = gi = 0.0
    for j in range(8):
        gu += gu2[:, j * GD:(j + 1) * GD] * ohu[:, j:j + 1]
        gi += gi2[:, j * GD:(j + 1) * GD] * ohi[:, j:j + 1]
    g = gu * gi
    out[...] = (jnp.dot(g, wfg[...], preferred_element_type=f32)
                + jnp.dot(mlp, wfm[...], preferred_element_type=f32))


def kernel(user_index, item_index, user_gmf, user_mlp, item_gmf, item_mlp,
           W1, b1, W2, b2, W3, b3, Wf):
    B = user_index.shape[0]
    ui = user_index.astype(jnp.int32)
    ii = item_index.astype(jnp.int32)
    U = user_mlp.shape[0]
    V = item_mlp.shape[0]

    umlp_p = user_mlp.reshape(U * D // W, W)
    imlp_p = item_mlp.reshape(V * D // W, W)
    ugmf_p = user_gmf.reshape(U * GD // W, W)
    igmf_p = item_gmf.reshape(V * GD // W, W)

    shape3 = (NW, -1, CHUNK)
    uh2, ih2, gu2, gi2 = _make_gather(B)(
        (ui >> 1).reshape(shape3), (ii >> 1).reshape(shape3),
        (ui >> 3).reshape(shape3), (ii >> 3).reshape(shape3),
        umlp_p, imlp_p, ugmf_p, igmf_p)

    pu = (ui & 1).astype(jnp.float32).reshape(B, 1)
    pi = (ii & 1).astype(jnp.float32).reshape(B, 1)
    ohu = jax.nn.one_hot(ui & 7, 8, dtype=jnp.float32)
    ohi = jax.nn.one_hot(ii & 7, 8, dtype=jnp.float32)

    TB = 2048
    grid = (B // TB,)
    row = lambda i: (i, 0)
    rep = lambda i: (0, 0)
    out = pl.pallas_call(
        _mlp_body,
        grid=grid,
        in_specs=[
            pl.BlockSpec((TB, W), row),
            pl.BlockSpec((TB, W), row),
            pl.BlockSpec((TB, W), row),
            pl.BlockSpec((TB, W), row),
            pl.BlockSpec((TB, 1), row),
            pl.BlockSpec((TB, 1), row),
            pl.BlockSpec((TB, 8), row),
            pl.BlockSpec((TB, 8), row),
            pl.BlockSpec((D, D), rep),
            pl.BlockSpec((D, D), rep),
            pl.BlockSpec((1, D), rep),
            pl.BlockSpec((D, H2), rep),
            pl.BlockSpec((1, H2), rep),
            pl.BlockSpec((H2, GD), rep),
            pl.BlockSpec((1, GD), rep),
            pl.BlockSpec((GD, 1), rep),
            pl.BlockSpec((GD, 1), rep),
        ],
        out_specs=pl.BlockSpec((TB, 1), row),
        out_shape=jax.ShapeDtypeStruct((B, 1), jnp.float32),
    )(uh2, ih2, gu2, gi2, pu, pi, ohu, ohi,
      W1[:D], W1[D:], b1.reshape(1, D),
      W2, b2.reshape(1, H2),
      W3, b3.reshape(1, GD),
      Wf[:GD].reshape(GD, 1), Wf[GD:].reshape(GD, 1))
    return out.reshape(B)


# speedup vs baseline: 1.0472x; 1.0472x over previous
import functools

import jax
import jax.numpy as jnp
from jax import lax
from jax.experimental import pallas as pl
from jax.experimental.pallas import tpu as pltpu
from jax.experimental.pallas import tpu_sc as plsc

D = 64
GD = D // 4
H2 = D // 2
NC, NS = 2, 16
NW = NC * NS
CHUNK = 128
W = 128


def _pack_mlp_body(xu, xi, yu, yi):
    for a in range(4):
        for x, y in ((xu, yu), (xi, yi)):
            y[pl.ds(128 * a, 128), :D] = x[:, pl.ds(256 * a, 128)].T
            y[pl.ds(128 * a, 128), D:] = x[:, pl.ds(256 * a + 128, 128)].T


def _pack_gmf_body(xu, xi, yu, yi):
    for a in range(8):
        for x, y in ((xu, yu), (xi, yi)):
            y[:, pl.ds(GD * a, GD)] = x[:, pl.ds(128 * a, 128)].T


def _pack(body, tab_u, tab_i, nfeat, rows_out, cols_in):
    nblk = (tab_u.shape[1] + cols_in - 1) // cols_in
    return pl.pallas_call(
        body,
        grid=(nblk,),
        in_specs=[pl.BlockSpec((nfeat, cols_in), lambda i: (0, i))] * 2,
        out_specs=[pl.BlockSpec((rows_out, W), lambda i: (i, 0))] * 2,
        out_shape=[jax.ShapeDtypeStruct((nblk * rows_out, W), jnp.float32)] * 2,
    )(tab_u, tab_i)


@functools.lru_cache(maxsize=None)
def _make_gather(B):
    b_per_w = B // NW
    n_chunks = b_per_w // CHUNK
    mesh = plsc.VectorSubcoreMesh(core_axis_name="c", subcore_axis_name="s")

    @functools.partial(
        pl.kernel,
        mesh=mesh,
        out_type=[jax.ShapeDtypeStruct((B, W), jnp.float32) for _ in range(4)],
        scratch_types=[
            pltpu.VMEM((n_chunks, CHUNK), jnp.int32),
            pltpu.VMEM((n_chunks, CHUNK), jnp.int32),
            pltpu.VMEM((n_chunks, CHUNK), jnp.int32),
            pltpu.VMEM((n_chunks, CHUNK), jnp.int32),
            pltpu.VMEM((2, CHUNK, W), jnp.float32),
            pltpu.VMEM((2, CHUNK, W), jnp.float32),
            pltpu.SemaphoreType.DMA,
        ],
    )
    def gather(pu_hbm, pi_hbm, pgu_hbm, pgi_hbm, umlp, imlp, ugmf, igmf,
               uh_out, ih_out, gu_out, gi_out,
               pu_v, pi_v, pgu_v, pgi_v, ba, bb, sem):
        wid = lax.axis_index("s") * NC + lax.axis_index("c")
        base = wid * b_per_w
        pltpu.sync_copy(pu_hbm.at[wid], pu_v)
        pltpu.sync_copy(pi_hbm.at[wid], pi_v)
        pltpu.sync_copy(pgu_hbm.at[wid], pgu_v)
        pltpu.sync_copy(pgi_hbm.at[wid], pgi_v)

        def run_pair(tab_a, idx_a, out_a, tab_b, idx_b, out_b):
            def fire(c, bank):
                pltpu.async_copy(tab_a.at[idx_a.at[c]], ba.at[bank], sem)
                pltpu.async_copy(tab_b.at[idx_b.at[c]], bb.at[bank], sem)

            def drain(bank):
                pltpu.make_async_copy(tab_a.at[idx_a.at[0]], ba.at[bank],
                                      sem).wait()
                pltpu.make_async_copy(tab_a.at[idx_a.at[0]], bb.at[bank],
                                      sem).wait()

            fire(0, 0)
            for c in range(n_chunks):
                bank = c % 2
                drain(bank)
                if c + 1 < n_chunks:
                    fire(c + 1, 1 - bank)
                sl = pl.ds(base + c * CHUNK, CHUNK)
                pltpu.sync_copy(ba.at[bank], out_a.at[sl])
                pltpu.sync_copy(bb.at[bank], out_b.at[sl])

        run_pair(umlp, pu_v, uh_out, imlp, pi_v, ih_out)
        run_pair(ugmf, pgu_v, gu_out, igmf, pgi_v, gi_out)

    return gather


def _mlp_body(uh2, ih2, gu2, gi2, pu, pi, ohu, ohi,
              w1u, w1i, b1, w2, b2, w3, b3, wfg, wfm, out):
    f32 = jnp.float32
    uh = jnp.where(pu[...] > 0, uh2[:, D:], uh2[:, :D])
    ih = jnp.where(pi[...] > 0, ih2[:, D:], ih2[:, :D])
    h1 = jnp.dot(uh, w1u[...], preferred_element_type=f32)
    h1 += jnp.dot(ih, w1i[...], preferred_element_type=f32) + b1[...]
    h1 = jnp.maximum(h1, 0.0)
    h2 = jnp.maximum(jnp.dot(h1, w2[...], preferred_element_type=f32) + b2[...], 0.0)
    mlp = jnp.dot(h2, w3[...], preferred_element_type=f32) + b3[...]
    gu = gi = 0.0
    for j in range(8):
        gu += gu2[:, j * GD:(j + 1) * GD] * ohu[:, j:j + 1]
        gi += gi2[:, j * GD:(j + 1) * GD] * ohi[:, j:j + 1]
    g = gu * gi
    out[...] = (jnp.dot(g, wfg[...], preferred_element_type=f32)
                + jnp.dot(mlp, wfm[...], preferred_element_type=f32))


def kernel(user_index, item_index, user_gmf, user_mlp, item_gmf, item_mlp,
           W1, b1, W2, b2, W3, b3, Wf):
    B = user_index.shape[0]
    ui = user_index.astype(jnp.int32)
    ii = item_index.astype(jnp.int32)
    U = user_mlp.shape[0]
    V = item_mlp.shape[0]

    umlp_p, imlp_p = _pack(_pack_mlp_body, user_mlp.T, item_mlp.T, D, 512, 1024)
    ugmf_p, igmf_p = _pack(_pack_gmf_body, user_gmf.T, item_gmf.T, GD, 128, 1024)

    shape3 = (NW, -1, CHUNK)
    pmu = ((ui >> 8) << 7) + (ui & 127)
    pmi = ((ii >> 8) << 7) + (ii & 127)
    pgu = ((ui >> 10) << 7) + (ui & 127)
    pgi = ((ii >> 10) << 7) + (ii & 127)
    uh2, ih2, gu2, gi2 = _make_gather(B)(
        pmu.reshape(shape3), pmi.reshape(shape3),
        pgu.reshape(shape3), pgi.reshape(shape3),
        umlp_p, imlp_p, ugmf_p, igmf_p)

    pu = ((ui >> 7) & 1).astype(jnp.float32).reshape(B, 1)
    pi = ((ii >> 7) & 1).astype(jnp.float32).reshape(B, 1)
    ohu = jax.nn.one_hot((ui >> 7) & 7, 8, dtype=jnp.float32)
    ohi = jax.nn.one_hot((ii >> 7) & 7, 8, dtype=jnp.float32)

    TB = 2048
    grid = (B // TB,)
    row = lambda i: (i, 0)
    rep = lambda i: (0, 0)
    out = pl.pallas_call(
        _mlp_body,
        grid=grid,
        in_specs=[
            pl.BlockSpec((TB, W), row),
            pl.BlockSpec((TB, W), row),
            pl.BlockSpec((TB, W), row),
            pl.BlockSpec((TB, W), row),
            pl.BlockSpec((TB, 1), row),
            pl.BlockSpec((TB, 1), row),
            pl.BlockSpec((TB, 8), row),
            pl.BlockSpec((TB, 8), row),
            pl.BlockSpec((D, D), rep),
            pl.BlockSpec((D, D), rep),
            pl.BlockSpec((1, D), rep),
            pl.BlockSpec((D, H2), rep),
            pl.BlockSpec((1, H2), rep),
            pl.BlockSpec((H2, GD), rep),
            pl.BlockSpec((1, GD), rep),
            pl.BlockSpec((GD, 1), rep),
            pl.BlockSpec((GD, 1), rep),
        ],
        out_specs=pl.BlockSpec((TB, 1), row),
        out_shape=jax.ShapeDtypeStruct((B, 1), jnp.float32),
    )(uh2, ih2, gu2, gi2, pu, pi, ohu, ohi,
      W1[:D], W1[D:], b1.reshape(1, D),
      W2, b2.reshape(1, H2),
      W3, b3.reshape(1, GD),
      Wf[:GD].reshape(GD, 1), Wf[GD:].reshape(GD, 1))
    return out.reshape(B)


# speedup vs baseline: 1.1607x; 1.1084x over previous
import functools

import jax
import jax.numpy as jnp
from jax import lax
from jax.experimental import pallas as pl
from jax.experimental.pallas import tpu as pltpu
from jax.experimental.pallas import tpu_sc as plsc

D = 64
GD = D // 4
H2 = D // 2
NC, NS = 2, 16
NW = NC * NS
CHUNK = 128
W = 128


def _t(eye, x):
    return lax.dot_general(eye[...], x, (((1,), (1,)), ((), ())),
                           preferred_element_type=jnp.float32)


def _pack_mlp_body(eye, xu, xi, yu, yi):
    for a in range(4):
        for x, y in ((xu, yu), (xi, yi)):
            y[pl.ds(128 * a, 128), :D] = _t(eye, x[:, pl.ds(256 * a, 128)])
            y[pl.ds(128 * a, 128), D:] = _t(eye, x[:, pl.ds(256 * a + 128, 128)])


def _pack_gmf_body(eye, xu, xi, yu, yi):
    for a in range(8):
        for x, y in ((xu, yu), (xi, yi)):
            y[:, pl.ds(GD * a, GD)] = _t(eye, x[:, pl.ds(128 * a, 128)])


def _pack(body, tab_u, tab_i, nfeat, rows_out, cols_in):
    nblk = (tab_u.shape[1] + cols_in - 1) // cols_in
    eye = jnp.eye(W, dtype=jnp.float32)
    return pl.pallas_call(
        body,
        grid=(nblk,),
        in_specs=[pl.BlockSpec((W, W), lambda i: (0, 0))]
        + [pl.BlockSpec((nfeat, cols_in), lambda i: (0, i))] * 2,
        out_specs=[pl.BlockSpec((rows_out, W), lambda i: (i, 0))] * 2,
        out_shape=[jax.ShapeDtypeStruct((nblk * rows_out, W), jnp.float32)] * 2,
    )(eye, tab_u, tab_i)


@functools.lru_cache(maxsize=None)
def _make_gather(B):
    b_per_w = B // NW
    n_chunks = b_per_w // CHUNK
    mesh = plsc.VectorSubcoreMesh(core_axis_name="c", subcore_axis_name="s")

    @functools.partial(
        pl.kernel,
        mesh=mesh,
        out_type=[jax.ShapeDtypeStruct((B, W), jnp.float32) for _ in range(4)],
        scratch_types=[
            pltpu.VMEM((n_chunks, CHUNK), jnp.int32),
            pltpu.VMEM((n_chunks, CHUNK), jnp.int32),
            pltpu.VMEM((n_chunks, CHUNK), jnp.int32),
            pltpu.VMEM((n_chunks, CHUNK), jnp.int32),
            pltpu.VMEM((2, CHUNK, W), jnp.float32),
            pltpu.VMEM((2, CHUNK, W), jnp.float32),
            pltpu.SemaphoreType.DMA,
        ],
    )
    def gather(pu_hbm, pi_hbm, pgu_hbm, pgi_hbm, umlp, imlp, ugmf, igmf,
               uh_out, ih_out, gu_out, gi_out,
               pu_v, pi_v, pgu_v, pgi_v, ba, bb, sem):
        wid = lax.axis_index("s") * NC + lax.axis_index("c")
        base = wid * b_per_w
        pltpu.sync_copy(pu_hbm.at[wid], pu_v)
        pltpu.sync_copy(pi_hbm.at[wid], pi_v)
        pltpu.sync_copy(pgu_hbm.at[wid], pgu_v)
        pltpu.sync_copy(pgi_hbm.at[wid], pgi_v)

        def run_pair(tab_a, idx_a, out_a, tab_b, idx_b, out_b):
            def fire(c, bank):
                pltpu.async_copy(tab_a.at[idx_a.at[c]], ba.at[bank], sem)
                pltpu.async_copy(tab_b.at[idx_b.at[c]], bb.at[bank], sem)

            def drain(bank):
                pltpu.make_async_copy(tab_a.at[idx_a.at[0]], ba.at[bank],
                                      sem).wait()
                pltpu.make_async_copy(tab_a.at[idx_a.at[0]], bb.at[bank],
                                      sem).wait()

            fire(0, 0)
            for c in range(n_chunks):
                bank = c % 2
                drain(bank)
                if c + 1 < n_chunks:
                    fire(c + 1, 1 - bank)
                sl = pl.ds(base + c * CHUNK, CHUNK)
                pltpu.sync_copy(ba.at[bank], out_a.at[sl])
                pltpu.sync_copy(bb.at[bank], out_b.at[sl])

        run_pair(umlp, pu_v, uh_out, imlp, pi_v, ih_out)
        run_pair(ugmf, pgu_v, gu_out, igmf, pgi_v, gi_out)

    return gather


def _mlp_body(uh2, ih2, gu2, gi2, hu, hi, su, si,
              w1uu, w1ii, b1, w2, b2, w3, b3, s16, wfg, wfm, out):
    f32 = jnp.float32
    tb = uh2.shape[0]
    c = lax.broadcasted_iota(jnp.int32, (tb, W), 1)
    mu = ((c >= D) == (hu[...] > 0)).astype(f32)
    mi = ((c >= D) == (hi[...] > 0)).astype(f32)
    ou = ((c >> 4) == su[...]).astype(f32)
    oi = ((c >> 4) == si[...]).astype(f32)
    h1 = jnp.dot(uh2[...] * mu, w1uu[...], preferred_element_type=f32)
    h1 += jnp.dot(ih2[...] * mi, w1ii[...], preferred_element_type=f32) + b1[...]
    h1 = jnp.maximum(h1, 0.0)
    h2 = jnp.maximum(jnp.dot(h1, w2[...], preferred_element_type=f32) + b2[...], 0.0)
    mlp = jnp.dot(h2, w3[...], preferred_element_type=f32) + b3[...]
    gu = jnp.dot(gu2[...] * ou, s16[...], preferred_element_type=f32)
    gi = jnp.dot(gi2[...] * oi, s16[...], preferred_element_type=f32)
    g = gu * gi
    out[...] = (jnp.dot(g, wfg[...], preferred_element_type=f32)
                + jnp.dot(mlp, wfm[...], preferred_element_type=f32))


def kernel(user_index, item_index, user_gmf, user_mlp, item_gmf, item_mlp,
           W1, b1, W2, b2, W3, b3, Wf):
    B = user_index.shape[0]
    ui = user_index.astype(jnp.int32)
    ii = item_index.astype(jnp.int32)
    U = user_mlp.shape[0]
    V = item_mlp.shape[0]

    umlp_p, imlp_p = _pack(_pack_mlp_body, user_mlp.T, item_mlp.T, D, 512, 1024)
    ugmf_p, igmf_p = _pack(_pack_gmf_body, user_gmf.T, item_gmf.T, GD, 128, 1024)

    shape3 = (NW, -1, CHUNK)
    pmu = ((ui >> 8) << 7) + (ui & 127)
    pmi = ((ii >> 8) << 7) + (ii & 127)
    pgu = ((ui >> 10) << 7) + (ui & 127)
    pgi = ((ii >> 10) << 7) + (ii & 127)
    uh2, ih2, gu2, gi2 = _make_gather(B)(
        pmu.reshape(shape3), pmi.reshape(shape3),
        pgu.reshape(shape3), pgi.reshape(shape3),
        umlp_p, imlp_p, ugmf_p, igmf_p)

    hu = ((ui >> 7) & 1).reshape(B, 1)
    hi = ((ii >> 7) & 1).reshape(B, 1)
    su = ((ui >> 7) & 7).reshape(B, 1)
    si = ((ii >> 7) & 7).reshape(B, 1)
    s16 = jnp.tile(jnp.eye(GD, dtype=jnp.float32), (8, 1))
    w1uu = jnp.concatenate([W1[:D], W1[:D]], axis=0)
    w1ii = jnp.concatenate([W1[D:], W1[D:]], axis=0)

    TB = 2048
    grid = (B // TB,)
    row = lambda i: (i, 0)
    rep = lambda i: (0, 0)
    out = pl.pallas_call(
        _mlp_body,
        grid=grid,
        in_specs=[
            pl.BlockSpec((TB, W), row),
            pl.BlockSpec((TB, W), row),
            pl.BlockSpec((TB, W), row),
            pl.BlockSpec((TB, W), row),
            pl.BlockSpec((TB, 1), row),
            pl.BlockSpec((TB, 1), row),
            pl.BlockSpec((TB, 1), row),
            pl.BlockSpec((TB, 1), row),
            pl.BlockSpec((W, D), rep),
            pl.BlockSpec((W, D), rep),
            pl.BlockSpec((1, D), rep),
            pl.BlockSpec((D, H2), rep),
            pl.BlockSpec((1, H2), rep),
            pl.BlockSpec((H2, GD), rep),
            pl.BlockSpec((1, GD), rep),
            pl.BlockSpec((W, GD), rep),
            pl.BlockSpec((GD, 1), rep),
            pl.BlockSpec((GD, 1), rep),
        ],
        out_specs=pl.BlockSpec((TB, 1), row),
        out_shape=jax.ShapeDtypeStruct((B, 1), jnp.float32),
    )(uh2, ih2, gu2, gi2, hu, hi, su, si,
      w1uu, w1ii, b1.reshape(1, D),
      W2, b2.reshape(1, H2),
      W3, b3.reshape(1, GD),
      s16, Wf[:GD].reshape(GD, 1), Wf[GD:].reshape(GD, 1))
    return out.reshape(B)


# speedup vs baseline: 2.6737x; 2.3034x over previous
import functools

import jax
import jax.numpy as jnp
from jax import lax
from jax.experimental import pallas as pl
from jax.experimental.pallas import tpu as pltpu
from jax.experimental.pallas import tpu_sc as plsc

D = 64
GD = D // 4
H2 = D // 2
NC, NS = 2, 16
NW = NC * NS
CHUNK = 128
W = 128


def _t(eye, x):
    return lax.dot_general(eye[...], x, (((1,), (1,)), ((), ())),
                           preferred_element_type=jnp.float32)


PC = 8192


def _pack_mlp_body(eye, xu, xi, yu, yi):
    for a in range(PC // 256):
        for x, y in ((xu, yu), (xi, yi)):
            y[pl.ds(128 * a, 128), :D] = _t(eye, x[:, pl.ds(256 * a, 128)])
            y[pl.ds(128 * a, 128), D:] = _t(eye, x[:, pl.ds(256 * a + 128, 128)])


def _pack_gmf_body(eye, xu, xi, yu, yi):
    for g in range(PC // 1024):
        for a in range(8):
            for x, y in ((xu, yu), (xi, yi)):
                y[pl.ds(128 * g, 128), pl.ds(GD * a, GD)] = _t(
                    eye, x[:, pl.ds(1024 * g + 128 * a, 128)])


def _pack(body, tab_u, tab_i, nfeat, rows_out):
    nblk = (tab_u.shape[1] + PC - 1) // PC
    eye = jnp.eye(W, dtype=jnp.float32)
    return pl.pallas_call(
        body,
        grid=(nblk,),
        in_specs=[pl.BlockSpec((W, W), lambda i: (0, 0))]
        + [pl.BlockSpec((nfeat, PC), lambda i: (0, i))] * 2,
        out_specs=[pl.BlockSpec((rows_out, W), lambda i: (i, 0))] * 2,
        out_shape=[jax.ShapeDtypeStruct((nblk * rows_out, W), jnp.float32)] * 2,
    )(eye, tab_u, tab_i)


@functools.lru_cache(maxsize=None)
def _make_gather(B):
    b_per_w = B // NW
    n_chunks = b_per_w // CHUNK
    mesh = plsc.VectorSubcoreMesh(core_axis_name="c", subcore_axis_name="s")

    @functools.partial(
        pl.kernel,
        mesh=mesh,
        out_type=[jax.ShapeDtypeStruct((B, W), jnp.float32) for _ in range(4)],
        scratch_types=[
            pltpu.VMEM((n_chunks, CHUNK), jnp.int32),
            pltpu.VMEM((n_chunks, CHUNK), jnp.int32),
            pltpu.VMEM((n_chunks, CHUNK), jnp.int32),
            pltpu.VMEM((n_chunks, CHUNK), jnp.int32),
            pltpu.VMEM((2, CHUNK, W), jnp.float32),
            pltpu.VMEM((2, CHUNK, W), jnp.float32),
            pltpu.SemaphoreType.DMA,
        ],
    )
    def gather(pu_hbm, pi_hbm, pgu_hbm, pgi_hbm, umlp, imlp, ugmf, igmf,
               uh_out, ih_out, gu_out, gi_out,
               pu_v, pi_v, pgu_v, pgi_v, ba, bb, sem):
        wid = lax.axis_index("s") * NC + lax.axis_index("c")
        base = wid * b_per_w
        pltpu.sync_copy(pu_hbm.at[wid], pu_v)
        pltpu.sync_copy(pi_hbm.at[wid], pi_v)
        pltpu.sync_copy(pgu_hbm.at[wid], pgu_v)
        pltpu.sync_copy(pgi_hbm.at[wid], pgi_v)

        def run_pair(tab_a, idx_a, out_a, tab_b, idx_b, out_b):
            def fire(c, bank):
                pltpu.async_copy(tab_a.at[idx_a.at[c]], ba.at[bank], sem)
                pltpu.async_copy(tab_b.at[idx_b.at[c]], bb.at[bank], sem)

            def drain(bank):
                pltpu.make_async_copy(tab_a.at[idx_a.at[0]], ba.at[bank],
                                      sem).wait()
                pltpu.make_async_copy(tab_a.at[idx_a.at[0]], bb.at[bank],
                                      sem).wait()

            fire(0, 0)
            for c in range(n_chunks):
                bank = c % 2
                drain(bank)
                if c + 1 < n_chunks:
                    fire(c + 1, 1 - bank)
                sl = pl.ds(base + c * CHUNK, CHUNK)
                pltpu.sync_copy(ba.at[bank], out_a.at[sl])
                pltpu.sync_copy(bb.at[bank], out_b.at[sl])

        run_pair(umlp, pu_v, uh_out, imlp, pi_v, ih_out)
        run_pair(ugmf, pgu_v, gu_out, igmf, pgi_v, gi_out)

    return gather


def _mlp_body(uh2, ih2, gu2, gi2, hu, hi, su, si,
              w1uu, w1ii, b1, w2, b2, w3, b3, s16, wfg, wfm, out):
    f32 = jnp.float32
    tb = uh2.shape[0]
    c = lax.broadcasted_iota(jnp.int32, (tb, W), 1)
    mu = ((c >= D) == (hu[...] > 0)).astype(f32)
    mi = ((c >= D) == (hi[...] > 0)).astype(f32)
    ou = ((c >> 4) == su[...]).astype(f32)
    oi = ((c >> 4) == si[...]).astype(f32)
    h1 = jnp.dot(uh2[...] * mu, w1uu[...], preferred_element_type=f32)
    h1 += jnp.dot(ih2[...] * mi, w1ii[...], preferred_element_type=f32) + b1[...]
    h1 = jnp.maximum(h1, 0.0)
    h2 = jnp.maximum(jnp.dot(h1, w2[...], preferred_element_type=f32) + b2[...], 0.0)
    mlp = jnp.dot(h2, w3[...], preferred_element_type=f32) + b3[...]
    gu = jnp.dot(gu2[...] * ou, s16[...], preferred_element_type=f32)
    gi = jnp.dot(gi2[...] * oi, s16[...], preferred_element_type=f32)
    g = gu * gi
    out[...] = (jnp.dot(g, wfg[...], preferred_element_type=f32)
                + jnp.dot(mlp, wfm[...], preferred_element_type=f32))


def kernel(user_index, item_index, user_gmf, user_mlp, item_gmf, item_mlp,
           W1, b1, W2, b2, W3, b3, Wf):
    B = user_index.shape[0]
    ui = user_index.astype(jnp.int32)
    ii = item_index.astype(jnp.int32)
    U = user_mlp.shape[0]
    V = item_mlp.shape[0]

    umlp_p, imlp_p = _pack(_pack_mlp_body, user_mlp.T, item_mlp.T, D, PC // 2)
    ugmf_p, igmf_p = _pack(_pack_gmf_body, user_gmf.T, item_gmf.T, GD, PC // 8)

    shape3 = (NW, -1, CHUNK)
    pmu = ((ui >> 8) << 7) + (ui & 127)
    pmi = ((ii >> 8) << 7) + (ii & 127)
    pgu = ((ui >> 10) << 7) + (ui & 127)
    pgi = ((ii >> 10) << 7) + (ii & 127)
    uh2, ih2, gu2, gi2 = _make_gather(B)(
        pmu.reshape(shape3), pmi.reshape(shape3),
        pgu.reshape(shape3), pgi.reshape(shape3),
        umlp_p, imlp_p, ugmf_p, igmf_p)

    hu = ((ui >> 7) & 1).reshape(B, 1)
    hi = ((ii >> 7) & 1).reshape(B, 1)
    su = ((ui >> 7) & 7).reshape(B, 1)
    si = ((ii >> 7) & 7).reshape(B, 1)
    s16 = jnp.tile(jnp.eye(GD, dtype=jnp.float32), (8, 1))
    w1uu = jnp.concatenate([W1[:D], W1[:D]], axis=0)
    w1ii = jnp.concatenate([W1[D:], W1[D:]], axis=0)

    TB = 2048
    grid = (B // TB,)
    row = lambda i: (i, 0)
    rep = lambda i: (0, 0)
    out = pl.pallas_call(
        _mlp_body,
        grid=grid,
        in_specs=[
            pl.BlockSpec((TB, W), row),
            pl.BlockSpec((TB, W), row),
            pl.BlockSpec((TB, W), row),
            pl.BlockSpec((TB, W), row),
            pl.BlockSpec((TB, 1), row),
            pl.BlockSpec((TB, 1), row),
            pl.BlockSpec((TB, 1), row),
            pl.BlockSpec((TB, 1), row),
            pl.BlockSpec((W, D), rep),
            pl.BlockSpec((W, D), rep),
            pl.BlockSpec((1, D), rep),
            pl.BlockSpec((D, H2), rep),
            pl.BlockSpec((1, H2), rep),
            pl.BlockSpec((H2, GD), rep),
            pl.BlockSpec((1, GD), rep),
            pl.BlockSpec((W, GD), rep),
            pl.BlockSpec((GD, 1), rep),
            pl.BlockSpec((GD, 1), rep),
        ],
        out_specs=pl.BlockSpec((TB, 1), row),
        out_shape=jax.ShapeDtypeStruct((B, 1), jnp.float32),
    )(uh2, ih2, gu2, gi2, hu, hi, su, si,
      w1uu, w1ii, b1.reshape(1, D),
      W2, b2.reshape(1, H2),
      W3, b3.reshape(1, GD),
      s16, Wf[:GD].reshape(GD, 1), Wf[GD:].reshape(GD, 1))
    return out.reshape(B)


# speedup vs baseline: 2.7031x; 1.0110x over previous
import functools

import jax
import jax.numpy as jnp
from jax import lax
from jax.experimental import pallas as pl
from jax.experimental.pallas import tpu as pltpu
from jax.experimental.pallas import tpu_sc as plsc

D = 64
GD = D // 4
H2 = D // 2
NC, NS = 2, 16
NW = NC * NS
CHUNK = 128
W = 128


def _t(eye, x):
    return lax.dot_general(eye[...], x, (((1,), (1,)), ((), ())),
                           preferred_element_type=jnp.float32)


PC = 8192


def _pack_mlp_body(eye, xu, xi, yu, yi):
    for a in range(PC // 256):
        for x, y in ((xu, yu), (xi, yi)):
            y[pl.ds(128 * a, 128), :D] = _t(eye, x[:, pl.ds(256 * a, 128)])
            y[pl.ds(128 * a, 128), D:] = _t(eye, x[:, pl.ds(256 * a + 128, 128)])


def _pack_gmf_body(eye, xu, xi, yu, yi):
    for g in range(PC // 1024):
        for a in range(8):
            for x, y in ((xu, yu), (xi, yi)):
                y[pl.ds(128 * g, 128), pl.ds(GD * a, GD)] = _t(
                    eye, x[:, pl.ds(1024 * g + 128 * a, 128)])


def _pack(body, tab_u, tab_i, nfeat, rows_out):
    nblk = (tab_u.shape[1] + PC - 1) // PC
    eye = jnp.eye(W, dtype=jnp.float32)
    return pl.pallas_call(
        body,
        grid=(nblk,),
        in_specs=[pl.BlockSpec((W, W), lambda i: (0, 0))]
        + [pl.BlockSpec((nfeat, PC), lambda i: (0, i))] * 2,
        out_specs=[pl.BlockSpec((rows_out, W), lambda i: (i, 0))] * 2,
        out_shape=[jax.ShapeDtypeStruct((nblk * rows_out, W), jnp.float32)] * 2,
    )(eye, tab_u, tab_i)


@functools.lru_cache(maxsize=None)
def _make_gather(B):
    b_per_w = B // NW
    n_chunks = b_per_w // CHUNK
    mesh = plsc.VectorSubcoreMesh(core_axis_name="c", subcore_axis_name="s")

    @functools.partial(
        pl.kernel,
        mesh=mesh,
        out_type=[jax.ShapeDtypeStruct((B, W), jnp.float32) for _ in range(2)],
        scratch_types=[
            pltpu.VMEM((n_chunks, CHUNK), jnp.int32),
            pltpu.VMEM((n_chunks, CHUNK), jnp.int32),
            pltpu.VMEM((2, CHUNK, W), jnp.float32),
            pltpu.VMEM((2, CHUNK, W), jnp.float32),
            pltpu.SemaphoreType.DMA,
        ],
    )
    def gather(pu_hbm, pi_hbm, tab_u, tab_i, u_out, i_out,
               pu_v, pi_v, ba, bb, sem):
        wid = lax.axis_index("s") * NC + lax.axis_index("c")
        base = wid * b_per_w
        pltpu.sync_copy(pu_hbm.at[wid], pu_v)
        pltpu.sync_copy(pi_hbm.at[wid], pi_v)

        def fire(c, bank):
            pltpu.async_copy(tab_u.at[pu_v.at[c]], ba.at[bank], sem)
            pltpu.async_copy(tab_i.at[pi_v.at[c]], bb.at[bank], sem)

        def drain(bank):
            pltpu.make_async_copy(tab_u.at[pu_v.at[0]], ba.at[bank], sem).wait()
            pltpu.make_async_copy(tab_u.at[pu_v.at[0]], bb.at[bank], sem).wait()

        fire(0, 0)
        for c in range(n_chunks):
            bank = c % 2
            drain(bank)
            if c + 1 < n_chunks:
                fire(c + 1, 1 - bank)
            sl = pl.ds(base + c * CHUNK, CHUNK)
            pltpu.sync_copy(ba.at[bank], u_out.at[sl])
            pltpu.sync_copy(bb.at[bank], i_out.at[sl])

    return gather


def _mlp_body(uh2, ih2, gu2, gi2, hu, hi, su, si,
              w1uu, w1ii, b1, w2, b2, w3, b3, s16, wfg, wfm, out):
    f32 = jnp.float32
    tb = uh2.shape[0]
    c = lax.broadcasted_iota(jnp.int32, (tb, W), 1)
    mu = ((c >= D) == (hu[...] > 0)).astype(f32)
    mi = ((c >= D) == (hi[...] > 0)).astype(f32)
    ou = ((c >> 4) == su[...]).astype(f32)
    oi = ((c >> 4) == si[...]).astype(f32)
    h1 = jnp.dot(uh2[...] * mu, w1uu[...], preferred_element_type=f32)
    h1 += jnp.dot(ih2[...] * mi, w1ii[...], preferred_element_type=f32) + b1[...]
    h1 = jnp.maximum(h1, 0.0)
    h2 = jnp.maximum(jnp.dot(h1, w2[...], preferred_element_type=f32) + b2[...], 0.0)
    mlp = jnp.dot(h2, w3[...], preferred_element_type=f32) + b3[...]
    gu = jnp.dot(gu2[...] * ou, s16[...], preferred_element_type=f32)
    gi = jnp.dot(gi2[...] * oi, s16[...], preferred_element_type=f32)
    g = gu * gi
    out[...] = (jnp.dot(g, wfg[...], preferred_element_type=f32)
                + jnp.dot(mlp, wfm[...], preferred_element_type=f32))


def kernel(user_index, item_index, user_gmf, user_mlp, item_gmf, item_mlp,
           W1, b1, W2, b2, W3, b3, Wf):
    B = user_index.shape[0]
    ui = user_index.astype(jnp.int32)
    ii = item_index.astype(jnp.int32)
    U = user_mlp.shape[0]
    V = item_mlp.shape[0]

    shape3 = (NW, -1, CHUNK)
    pmu = ((ui >> 8) << 7) + (ui & 127)
    pmi = ((ii >> 8) << 7) + (ii & 127)
    pgu = ((ui >> 10) << 7) + (ui & 127)
    pgi = ((ii >> 10) << 7) + (ii & 127)

    umlp_p, imlp_p = _pack(_pack_mlp_body, user_mlp.T, item_mlp.T, D, PC // 2)
    uh2, ih2 = _make_gather(B)(pmu.reshape(shape3), pmi.reshape(shape3),
                               umlp_p, imlp_p)
    ugmf_p, igmf_p = _pack(_pack_gmf_body, user_gmf.T, item_gmf.T, GD, PC // 8)
    gu2, gi2 = _make_gather(B)(pgu.reshape(shape3), pgi.reshape(shape3),
                               ugmf_p, igmf_p)

    hu = ((ui >> 7) & 1).reshape(B, 1)
    hi = ((ii >> 7) & 1).reshape(B, 1)
    su = ((ui >> 7) & 7).reshape(B, 1)
    si = ((ii >> 7) & 7).reshape(B, 1)
    s16 = jnp.tile(jnp.eye(GD, dtype=jnp.float32), (8, 1))
    w1uu = jnp.concatenate([W1[:D], W1[:D]], axis=0)
    w1ii = jnp.concatenate([W1[D:], W1[D:]], axis=0)

    TB = 2048
    grid = (B // TB,)
    row = lambda i: (i, 0)
    rep = lambda i: (0, 0)
    out = pl.pallas_call(
        _mlp_body,
        grid=grid,
        in_specs=[
            pl.BlockSpec((TB, W), row),
            pl.BlockSpec((TB, W), row),
            pl.BlockSpec((TB, W), row),
            pl.BlockSpec((TB, W), row),
            pl.BlockSpec((TB, 1), row),
            pl.BlockSpec((TB, 1), row),
            pl.BlockSpec((TB, 1), row),
            pl.BlockSpec((TB, 1), row),
            pl.BlockSpec((W, D), rep),
            pl.BlockSpec((W, D), rep),
            pl.BlockSpec((1, D), rep),
            pl.BlockSpec((D, H2), rep),
            pl.BlockSpec((1, H2), rep),
            pl.BlockSpec((H2, GD), rep),
            pl.BlockSpec((1, GD), rep),
            pl.BlockSpec((W, GD), rep),
            pl.BlockSpec((GD, 1), rep),
            pl.BlockSpec((GD, 1), rep),
        ],
        out_specs=pl.BlockSpec((TB, 1), row),
        out_shape=jax.ShapeDtypeStruct((B, 1), jnp.float32),
    )(uh2, ih2, gu2, gi2, hu, hi, su, si,
      w1uu, w1ii, b1.reshape(1, D),
      W2, b2.reshape(1, H2),
      W3, b3.reshape(1, GD),
      s16, Wf[:GD].reshape(GD, 1), Wf[GD:].reshape(GD, 1))
    return out.reshape(B)


# speedup vs baseline: 2.7850x; 1.0303x over previous
import functools

import jax
import jax.numpy as jnp
from jax import lax
from jax.experimental import pallas as pl
from jax.experimental.pallas import tpu as pltpu
from jax.experimental.pallas import tpu_sc as plsc

D = 64
GD = D // 4
H2 = D // 2
NC, NS = 2, 16
NW = NC * NS
CHUNK = 128
W = 128


def _t(eye, x):
    return lax.dot_general(eye[...], x, (((1,), (1,)), ((), ())),
                           preferred_element_type=jnp.float32)


PC = 16384


def _pack_mlp_body(eye, xu, xi, yu, yi):
    for a in range(PC // 256):
        for x, y in ((xu, yu), (xi, yi)):
            y[pl.ds(128 * a, 128), :D] = _t(eye, x[:, pl.ds(256 * a, 128)])
            y[pl.ds(128 * a, 128), D:] = _t(eye, x[:, pl.ds(256 * a + 128, 128)])


def _pack_gmf_body(eye, xu, xi, yu, yi):
    for g in range(PC // 1024):
        for a in range(8):
            for x, y in ((xu, yu), (xi, yi)):
                y[pl.ds(128 * g, 128), pl.ds(GD * a, GD)] = _t(
                    eye, x[:, pl.ds(1024 * g + 128 * a, 128)])


def _pack(body, tab_u, tab_i, nfeat, rows_out):
    nblk = (tab_u.shape[1] + PC - 1) // PC
    eye = jnp.eye(W, dtype=jnp.float32)
    return pl.pallas_call(
        body,
        grid=(nblk,),
        in_specs=[pl.BlockSpec((W, W), lambda i: (0, 0))]
        + [pl.BlockSpec((nfeat, PC), lambda i: (0, i))] * 2,
        out_specs=[pl.BlockSpec((rows_out, W), lambda i: (i, 0))] * 2,
        out_shape=[jax.ShapeDtypeStruct((nblk * rows_out, W), jnp.float32)] * 2,
    )(eye, tab_u, tab_i)


@functools.lru_cache(maxsize=None)
def _make_gather(B):
    b_per_w = B // NW
    n_chunks = b_per_w // CHUNK
    mesh = plsc.VectorSubcoreMesh(core_axis_name="c", subcore_axis_name="s")

    @functools.partial(
        pl.kernel,
        mesh=mesh,
        out_type=[jax.ShapeDtypeStruct((B, W), jnp.float32) for _ in range(2)],
        scratch_types=[
            pltpu.VMEM((n_chunks, CHUNK), jnp.int32),
            pltpu.VMEM((n_chunks, CHUNK), jnp.int32),
            pltpu.VMEM((2, CHUNK, W), jnp.float32),
            pltpu.VMEM((2, CHUNK, W), jnp.float32),
            pltpu.SemaphoreType.DMA,
        ],
    )
    def gather(pu_hbm, pi_hbm, tab_u, tab_i, u_out, i_out,
               pu_v, pi_v, ba, bb, sem):
        wid = lax.axis_index("s") * NC + lax.axis_index("c")
        base = wid * b_per_w
        pltpu.sync_copy(pu_hbm.at[wid], pu_v)
        pltpu.sync_copy(pi_hbm.at[wid], pi_v)

        def fire(c, bank):
            pltpu.async_copy(tab_u.at[pu_v.at[c]], ba.at[bank], sem)
            pltpu.async_copy(tab_i.at[pi_v.at[c]], bb.at[bank], sem)

        def drain(bank):
            pltpu.make_async_copy(tab_u.at[pu_v.at[0]], ba.at[bank], sem).wait()
            pltpu.make_async_copy(tab_u.at[pu_v.at[0]], bb.at[bank], sem).wait()

        fire(0, 0)
        for c in range(n_chunks):
            bank = c % 2
            drain(bank)
            if c + 1 < n_chunks:
                fire(c + 1, 1 - bank)
            sl = pl.ds(base + c * CHUNK, CHUNK)
            pltpu.sync_copy(ba.at[bank], u_out.at[sl])
            pltpu.sync_copy(bb.at[bank], i_out.at[sl])

    return gather


def _mlp_body(uh2, ih2, gu2, gi2, hu, hi, su, si,
              w1uu, w1ii, b1, w2, b2, w3, b3, s16, wfg, wfm, out):
    f32 = jnp.float32
    tb = uh2.shape[0]
    c = lax.broadcasted_iota(jnp.int32, (tb, W), 1)
    mu = ((c >= D) == (hu[...] > 0)).astype(f32)
    mi = ((c >= D) == (hi[...] > 0)).astype(f32)
    ou = ((c >> 4) == su[...]).astype(f32)
    oi = ((c >> 4) == si[...]).astype(f32)
    h1 = jnp.dot(uh2[...] * mu, w1uu[...], preferred_element_type=f32)
    h1 += jnp.dot(ih2[...] * mi, w1ii[...], preferred_element_type=f32) + b1[...]
    h1 = jnp.maximum(h1, 0.0)
    h2 = jnp.maximum(jnp.dot(h1, w2[...], preferred_element_type=f32) + b2[...], 0.0)
    mlp = jnp.dot(h2, w3[...], preferred_element_type=f32) + b3[...]
    gu = jnp.dot(gu2[...] * ou, s16[...], preferred_element_type=f32)
    gi = jnp.dot(gi2[...] * oi, s16[...], preferred_element_type=f32)
    g = gu * gi
    out[...] = (jnp.dot(g, wfg[...], preferred_element_type=f32)
                + jnp.dot(mlp, wfm[...], preferred_element_type=f32))


def kernel(user_index, item_index, user_gmf, user_mlp, item_gmf, item_mlp,
           W1, b1, W2, b2, W3, b3, Wf):
    B = user_index.shape[0]
    ui = user_index.astype(jnp.int32)
    ii = item_index.astype(jnp.int32)
    U = user_mlp.shape[0]
    V = item_mlp.shape[0]

    shape3 = (NW, -1, CHUNK)
    pmu = ((ui >> 8) << 7) + (ui & 127)
    pmi = ((ii >> 8) << 7) + (ii & 127)
    pgu = ((ui >> 10) << 7) + (ui & 127)
    pgi = ((ii >> 10) << 7) + (ii & 127)

    umlp_p, imlp_p = _pack(_pack_mlp_body, user_mlp.T, item_mlp.T, D, PC // 2)
    uh2, ih2 = _make_gather(B)(pmu.reshape(shape3), pmi.reshape(shape3),
                               umlp_p, imlp_p)
    ugmf_p, igmf_p = _pack(_pack_gmf_body, user_gmf.T, item_gmf.T, GD, PC // 8)
    gu2, gi2 = _make_gather(B)(pgu.reshape(shape3), pgi.reshape(shape3),
                               ugmf_p, igmf_p)

    hu = ((ui >> 7) & 1).reshape(B, 1)
    hi = ((ii >> 7) & 1).reshape(B, 1)
    su = ((ui >> 7) & 7).reshape(B, 1)
    si = ((ii >> 7) & 7).reshape(B, 1)
    s16 = jnp.tile(jnp.eye(GD, dtype=jnp.float32), (8, 1))
    w1uu = jnp.concatenate([W1[:D], W1[:D]], axis=0)
    w1ii = jnp.concatenate([W1[D:], W1[D:]], axis=0)

    TB = 4096
    grid = (B // TB,)
    row = lambda i: (i, 0)
    rep = lambda i: (0, 0)
    out = pl.pallas_call(
        _mlp_body,
        grid=grid,
        in_specs=[
            pl.BlockSpec((TB, W), row),
            pl.BlockSpec((TB, W), row),
            pl.BlockSpec((TB, W), row),
            pl.BlockSpec((TB, W), row),
            pl.BlockSpec((TB, 1), row),
            pl.BlockSpec((TB, 1), row),
            pl.BlockSpec((TB, 1), row),
            pl.BlockSpec((TB, 1), row),
            pl.BlockSpec((W, D), rep),
            pl.BlockSpec((W, D), rep),
            pl.BlockSpec((1, D), rep),
            pl.BlockSpec((D, H2), rep),
            pl.BlockSpec((1, H2), rep),
            pl.BlockSpec((H2, GD), rep),
            pl.BlockSpec((1, GD), rep),
            pl.BlockSpec((W, GD), rep),
            pl.BlockSpec((GD, 1), rep),
            pl.BlockSpec((GD, 1), rep),
        ],
        out_specs=pl.BlockSpec((TB, 1), row),
        out_shape=jax.ShapeDtypeStruct((B, 1), jnp.float32),
    )(uh2, ih2, gu2, gi2, hu, hi, su, si,
      w1uu, w1ii, b1.reshape(1, D),
      W2, b2.reshape(1, H2),
      W3, b3.reshape(1, GD),
      s16, Wf[:GD].reshape(GD, 1), Wf[GD:].reshape(GD, 1))
    return out.reshape(B)
